# Initial kernel scaffold; baseline (speedup 1.0000x reference)
import jax
import jax.numpy as jnp
from jax.experimental import pallas as pl


def kernel(x, pe_table):
    raise NotImplementedError("write your pallas kernel here")


# speedup vs baseline: 1.4372x; 1.4372x over previous
import jax
import jax.numpy as jnp
from jax.experimental import pallas as pl


_BT = 512


def _add_pe_kernel(x_ref, pe_ref, o_ref):
    o_ref[...] = x_ref[...] + pe_ref[...][None, :, :]


def kernel(x, pe_table):
    B, T, C = x.shape
    grid = (T // _BT, B)
    return pl.pallas_call(
        _add_pe_kernel,
        grid=grid,
        in_specs=[
            pl.BlockSpec((1, _BT, C), lambda t, b: (b, t, 0)),
            pl.BlockSpec((_BT, C), lambda t, b: (t, 0)),
        ],
        out_specs=pl.BlockSpec((1, _BT, C), lambda t, b: (b, t, 0)),
        out_shape=jax.ShapeDtypeStruct((B, T, C), x.dtype),
    )(x, pe_table[:T])


# speedup vs baseline: 1.6851x; 1.1725x over previous
import jax
import jax.numpy as jnp
from jax.experimental import pallas as pl


_BT = 1024


def _add_pe_kernel(x_ref, pe_ref, o_ref):
    o_ref[...] = x_ref[...] + pe_ref[...][None, :, :]


def kernel(x, pe_table):
    B, T, C = x.shape
    grid = (T // _BT, B)
    return pl.pallas_call(
        _add_pe_kernel,
        grid=grid,
        in_specs=[
            pl.BlockSpec((1, _BT, C), lambda t, b: (b, t, 0)),
            pl.BlockSpec((_BT, C), lambda t, b: (t, 0)),
        ],
        out_specs=pl.BlockSpec((1, _BT, C), lambda t, b: (b, t, 0)),
        out_shape=jax.ShapeDtypeStruct((B, T, C), x.dtype),
    )(x, pe_table[:T])


# speedup vs baseline: 1.7904x; 1.0625x over previous
import jax
import jax.numpy as jnp
from jax.experimental import pallas as pl


_BT = 2048


def _add_pe_kernel(x_ref, pe_ref, o_ref):
    o_ref[...] = x_ref[...] + pe_ref[...][None, :, :]


def kernel(x, pe_table):
    B, T, C = x.shape
    grid = (T // _BT, B)
    return pl.pallas_call(
        _add_pe_kernel,
        grid=grid,
        in_specs=[
            pl.BlockSpec((1, _BT, C), lambda t, b: (b, t, 0)),
            pl.BlockSpec((_BT, C), lambda t, b: (t, 0)),
        ],
        out_specs=pl.BlockSpec((1, _BT, C), lambda t, b: (b, t, 0)),
        out_shape=jax.ShapeDtypeStruct((B, T, C), x.dtype),
    )(x, pe_table[:T])
